# baseline (speedup 1.0000x reference)
import jax
import jax.numpy as jnp
from jax import lax
from jax.experimental import pallas as pl
from jax.experimental.pallas import tpu as pltpu
from jax.experimental.pallas import tpu_sc as plsc

N_FIELDS = 26
VOCAB = 100000
BATCH = 16384
DIMS = 32

NC = 2
NS = 16
LANES = 16
NW = NC * NS
B_PER_W = BATCH // NW
WIN = 128
NWIN = B_PER_W // WIN
NWINDOWS = N_FIELDS * NWIN
NBUF = 2

PACK = 4
TP_VCH = 2048
TP_Q = TP_VCH // PACK
TP_GRID = -(-VOCAB // TP_VCH)
PROWS = TP_GRID * TP_Q


def _tc_pack_body(in_ref, out_ref):
    x = in_ref[0]
    y = jnp.transpose(x, (1, 0)) * (1.0 / N_FIELDS)
    out_ref[0] = jnp.concatenate(
        [y[q * TP_Q:(q + 1) * TP_Q, :] for q in range(PACK)], axis=1)


def _sc_body(tab_hbm, g_hbm, sub_hbm, out_hbm,
             g_v, sub_v, b0, b1, acc_v, s0, s1):
    bufs = (b0, b1)
    sems = (s0, s1)
    wid = lax.axis_index("s") * NC + lax.axis_index("c")

    pltpu.sync_copy(g_hbm.at[:, pl.ds(wid * NWIN, NWIN), :], g_v)
    pltpu.sync_copy(sub_hbm.at[:, pl.ds(wid * NWIN, NWIN), :], sub_v)

    zero = jnp.zeros((LANES,), jnp.float32)

    @pl.loop(0, B_PER_W)
    def _(r):
        acc_v[r, pl.ds(0, LANES)] = zero
        acc_v[r, pl.ds(LANES, LANES)] = zero

    for b in range(NBUF):
        pltpu.async_copy(tab_hbm.at[b // NWIN].at[g_v.at[b // NWIN, b % NWIN]],
                         bufs[b], sems[b])

    @pl.loop(0, NWINDOWS, step=NBUF)
    def _(i):
        for b in range(NBUF):
            buf, sem = bufs[b], sems[b]
            k = i + b
            f = k // NWIN
            h = k % NWIN
            pltpu.make_async_copy(tab_hbm.at[f].at[g_v.at[f, h]],
                                  buf, sem).wait()

            base = h * WIN

            @pl.loop(0, WIN, step=LANES)
            def _(r0, buf=buf, base=base, f=f, h=h):
                subvec = sub_v[f, h, pl.ds(r0, LANES)]
                for i in range(LANES):
                    sub = subvec[i]
                    plsc.addupdate(acc_v.at[base + r0 + i, pl.ds(0, LANES)],
                                   buf[r0 + i, pl.ds(sub, LANES)])
                    plsc.addupdate(
                        acc_v.at[base + r0 + i, pl.ds(LANES, LANES)],
                        buf[r0 + i, pl.ds(sub + LANES, LANES)])

            kn = k + NBUF

            @pl.when(kn < NWINDOWS)
            def _(buf=buf, sem=sem, kn=kn):
                fn = kn // NWIN
                hn = kn % NWIN
                pltpu.async_copy(tab_hbm.at[fn].at[g_v.at[fn, hn]], buf, sem)

    pltpu.sync_copy(acc_v, out_hbm.at[pl.ds(wid * B_PER_W, B_PER_W)])


@jax.jit
def _embedded_features(tables_t, cats):
    packed = pl.pallas_call(
        _tc_pack_body,
        out_shape=jax.ShapeDtypeStruct((N_FIELDS, PROWS, PACK * DIMS),
                                       jnp.float32),
        grid=(N_FIELDS, TP_GRID),
        in_specs=[pl.BlockSpec((1, DIMS, TP_VCH), lambda f, c: (f, 0, c))],
        out_specs=pl.BlockSpec((1, TP_Q, PACK * DIMS),
                               lambda f, c: (f, c, 0)),
    )(tables_t)

    g = (((cats >> 11) << 9) + (cats & 511)).reshape(N_FIELDS, NW * NWIN, WIN)
    sub = (((cats >> 9) & 3) << 5).reshape(N_FIELDS, NW * NWIN, WIN)

    mesh = plsc.VectorSubcoreMesh(core_axis_name="c", subcore_axis_name="s")
    k = pl.kernel(
        _sc_body,
        out_type=jax.ShapeDtypeStruct((BATCH, DIMS), jnp.float32),
        mesh=mesh,
        scratch_types=[
            pltpu.VMEM((N_FIELDS, NWIN, WIN), jnp.int32),
            pltpu.VMEM((N_FIELDS, NWIN, WIN), jnp.int32),
            pltpu.VMEM((WIN, PACK * DIMS), jnp.float32),
            pltpu.VMEM((WIN, PACK * DIMS), jnp.float32),
            pltpu.VMEM((B_PER_W, DIMS), jnp.float32),
            pltpu.SemaphoreType.DMA,
            pltpu.SemaphoreType.DMA,
        ],
        compiler_params=pltpu.CompilerParams(use_tc_tiling_on_sc=True),
    )
    return k(packed, g, sub)


def kernel(cats, tables):
    tables_t = jnp.transpose(tables, (0, 2, 1))
    return _embedded_features(tables_t, cats)
